# baseline (speedup 1.0000x reference)
import functools

import jax
import jax.numpy as jnp
from jax import lax
from jax.experimental import pallas as pl
from jax.experimental.pallas import tpu as pltpu
from jax.experimental.pallas import tpu_sc as plsc

D_MODEL = 128
NUM_WORKERS = 32
CHUNK = 128
UNIT = 2 * CHUNK
NBUF = 3


@functools.partial(jax.jit, static_argnums=(0,))
def _gather_rows(n_chunks, index_w, table):
    rows_total = NUM_WORKERS * n_chunks * CHUNK
    rows_per_w = n_chunks * CHUNK
    n_units = n_chunks // 2
    mesh = plsc.VectorSubcoreMesh(core_axis_name="c", subcore_axis_name="s")
    assert n_chunks % 2 == 0 and n_units >= 7 and (n_units - 4) % NBUF == 0

    @functools.partial(
        pl.kernel,
        mesh=mesh,
        out_type=jax.ShapeDtypeStruct((rows_total, D_MODEL), jnp.float32),
        scratch_types=[
            pltpu.VMEM((n_chunks, CHUNK), jnp.int32),
            pltpu.VMEM_SHARED((1000, D_MODEL), jnp.float32),
            *[pltpu.VMEM((UNIT, D_MODEL), jnp.float32) for _ in range(NBUF)],
            *[pltpu.SemaphoreType.DMA for _ in range(2 * NBUF)],
        ],
    )
    def k(idx_hbm, table_hbm, out_hbm, idx_v, table_sp, b0, b1, b2,
          g0, g1, g2, w0, w1, w2):
        bufs = (b0, b1, b2)
        gsems = (g0, g1, g2)
        wsems = (w0, w1, w2)
        wid = lax.axis_index("s") * 2 + lax.axis_index("c")
        base = wid * rows_per_w

        @pl.when(lax.axis_index("s") == 0)
        def _():
            pltpu.sync_copy(table_hbm, table_sp)

        pltpu.sync_copy(idx_hbm.at[wid], idx_v)
        plsc.subcore_barrier()

        def g_start(u, bi):
            for h in range(2):
                pltpu.async_copy(
                    table_sp.at[idx_v.at[2 * u + h]],
                    bufs[bi].at[pl.ds(h * CHUNK, CHUNK)], gsems[bi])

        def g_wait(bi):
            for _ in range(2):
                pltpu.make_async_copy(
                    table_sp.at[idx_v.at[0]],
                    bufs[bi].at[pl.ds(0, CHUNK)], gsems[bi]).wait()

        def w_start(u, bi):
            pltpu.async_copy(
                bufs[bi], out_hbm.at[pl.ds(base + u * UNIT, UNIT)], wsems[bi])

        def w_wait(bi):
            pltpu.make_async_copy(
                bufs[bi], out_hbm.at[pl.ds(base, UNIT)], wsems[bi]).wait()

        g_start(0, 0)
        for u in (0, 1):
            g_wait(u % NBUF)
            g_start(u + 1, (u + 1) % NBUF)
            w_start(u, u % NBUF)

        @pl.loop(2, n_units - 2, step=NBUF)
        def body(g):
            for b in range(NBUF):
                bi = (2 + b) % NBUF
                u = g + b
                g_wait(bi)
                w_wait((bi + 1) % NBUF)
                g_start(u + 1, (bi + 1) % NBUF)
                w_start(u, bi)

        u = n_units - 2
        g_wait(u % NBUF)
        w_wait((u + 1) % NBUF)
        g_start(u + 1, (u + 1) % NBUF)
        w_start(u, u % NBUF)
        u = n_units - 1
        g_wait(u % NBUF)
        w_wait((u + 1) % NBUF)
        w_start(u, u % NBUF)
        for u in (n_units - 2, n_units - 1):
            w_wait(u % NBUF)

    return k(index_w, table)


def kernel(patch_shape, index, position_embedding):
    batch, patch_num = index.shape
    d_model = position_embedding.shape[1]
    rows = batch * patch_num
    n_chunks = rows // (NUM_WORKERS * CHUNK)
    idx_t = index.astype(jnp.int32).T.reshape(NUM_WORKERS, n_chunks, CHUNK)
    out = _gather_rows(n_chunks, idx_t, position_embedding)
    return out.reshape(patch_num, batch, d_model).transpose(1, 0, 2)


# speedup vs baseline: 1.0335x; 1.0335x over previous
import functools

import jax
import jax.numpy as jnp
from jax import lax
from jax.experimental import pallas as pl
from jax.experimental.pallas import tpu as pltpu
from jax.experimental.pallas import tpu_sc as plsc

D_MODEL = 128
NUM_WORKERS = 32
CHUNK = 128
NBUF = 6
LOOK = 3


@functools.partial(jax.jit, static_argnums=(0,))
def _gather_rows(n_chunks, index_w, table):
    rows_total = NUM_WORKERS * n_chunks * CHUNK
    rows_per_w = n_chunks * CHUNK
    mesh = plsc.VectorSubcoreMesh(core_axis_name="c", subcore_axis_name="s")
    assert n_chunks >= NBUF + LOOK + 2
    assert (n_chunks - (2 * LOOK + 2)) % NBUF == 0

    @functools.partial(
        pl.kernel,
        mesh=mesh,
        out_type=jax.ShapeDtypeStruct((rows_total, D_MODEL), jnp.float32),
        scratch_types=[
            pltpu.VMEM((n_chunks, CHUNK), jnp.int32),
            pltpu.VMEM_SHARED((1000, D_MODEL), jnp.float32),
            *[pltpu.VMEM((CHUNK, D_MODEL), jnp.float32) for _ in range(NBUF)],
            *[pltpu.SemaphoreType.DMA for _ in range(2 * NBUF)],
        ],
    )
    def k(idx_hbm, table_hbm, out_hbm, idx_v, table_sp, *rest):
        bufs = rest[:NBUF]
        gsems = rest[NBUF:2 * NBUF]
        wsems = rest[2 * NBUF:]
        wid = lax.axis_index("s") * 2 + lax.axis_index("c")
        base = wid * rows_per_w

        @pl.when(lax.axis_index("s") == 0)
        def _():
            pltpu.sync_copy(table_hbm, table_sp)

        pltpu.sync_copy(idx_hbm.at[wid], idx_v)
        plsc.subcore_barrier()

        def g_start(gg, bi):
            pltpu.async_copy(table_sp.at[idx_v.at[gg]], bufs[bi], gsems[bi])

        def g_wait(bi):
            pltpu.make_async_copy(
                table_sp.at[idx_v.at[0]], bufs[bi], gsems[bi]).wait()

        def w_start(gg, bi):
            pltpu.async_copy(
                bufs[bi], out_hbm.at[pl.ds(base + gg * CHUNK, CHUNK)],
                wsems[bi])

        def w_wait(bi):
            pltpu.make_async_copy(
                bufs[bi], out_hbm.at[pl.ds(base, CHUNK)], wsems[bi]).wait()

        def enc(gg, bi):
            g_wait(bi)
            w_start(gg, bi)
            w_wait((bi + LOOK) % NBUF)
            g_start(gg + LOOK, (bi + LOOK) % NBUF)

        for gg in range(LOOK):
            g_start(gg, gg % NBUF)
        for gg in range(LOOK + 2):
            bi = gg % NBUF
            g_wait(bi)
            w_start(gg, bi)
            if gg >= NBUF - LOOK:
                w_wait((bi + LOOK) % NBUF)
            g_start(gg + LOOK, (gg + LOOK) % NBUF)

        @pl.loop(LOOK + 2, n_chunks - LOOK, step=NBUF)
        def body(g):
            for b in range(NBUF):
                enc(g + b, (LOOK + 2 + b) % NBUF)

        for gg in range(n_chunks - LOOK, n_chunks):
            bi = gg % NBUF
            g_wait(bi)
            w_start(gg, bi)
            w_wait((bi + LOOK) % NBUF)
        for gg in range(n_chunks - LOOK, n_chunks):
            w_wait(gg % NBUF)

    return k(index_w, table)


def kernel(patch_shape, index, position_embedding):
    batch, patch_num = index.shape
    d_model = position_embedding.shape[1]
    rows = batch * patch_num
    n_chunks = rows // (NUM_WORKERS * CHUNK)
    idx_t = index.astype(jnp.int32).T.reshape(NUM_WORKERS, n_chunks, CHUNK)
    out = _gather_rows(n_chunks, idx_t, position_embedding)
    return out.reshape(patch_num, batch, d_model).transpose(1, 0, 2)
